# baseline (speedup 1.0000x reference)
import functools

import jax
import jax.numpy as jnp
from jax import lax
from jax.experimental import pallas as pl
from jax.experimental.pallas import tpu as pltpu
from jax.experimental.pallas import tpu_sc as plsc

_B = 16384
_D = 64
_NCOL = 10
_NEG = 8
_NC = 2
_NS = 16
_NW = _NC * _NS
_BW = _B // _NW
_CH = 512
_NCHUNK = 2 + _BW * _NEG // _CH
_NBUF = 3
_L = 16


def _gather_triplets(items, table):
    mesh = plsc.VectorSubcoreMesh(core_axis_name="c", subcore_axis_name="s")

    @functools.partial(
        pl.kernel,
        mesh=mesh,
        out_type=(
            jax.ShapeDtypeStruct((_B, _D), jnp.float32),
            jax.ShapeDtypeStruct((_B, _D), jnp.float32),
            jax.ShapeDtypeStruct((_B * _NEG, _D), jnp.float32),
        ),
        scratch_types=(
            [
                pltpu.VMEM((_BW, _NCOL), jnp.int32),
                pltpu.VMEM((_BW,), jnp.int32),
                pltpu.VMEM((_BW,), jnp.int32),
                pltpu.VMEM((_BW * _NEG,), jnp.int32),
            ]
            + [pltpu.VMEM((_CH, _D), jnp.float32) for _ in range(_NBUF)]
            + [pltpu.SemaphoreType.DMA for _ in range(2 * _NBUF)]
        ),
        compiler_params=pltpu.CompilerParams(
            use_tc_tiling_on_sc=False, needs_layout_passes=False),
    )
    def body(items_hbm, table_hbm, vi_hbm, vk_hbm, vj_hbm, items_v,
             idxi_v, idxk_v, idxj_v, *rest):
        bufs = rest[:_NBUF]
        gsem = rest[_NBUF:2 * _NBUF]
        wsem = rest[2 * _NBUF:3 * _NBUF]

        wid = lax.axis_index("s") * _NC + lax.axis_index("c")
        base = wid * _BW
        jbase = wid * _BW * _NEG

        pltpu.sync_copy(items_hbm.at[pl.ds(base, _BW), :], items_v)

        lanes = lax.iota(jnp.int32, _L)

        def ik_step(s, _):
            rows = lanes + s * _L
            ti = plsc.load_gather(items_v, [rows, jnp.zeros_like(lanes)])
            ki = plsc.load_gather(items_v, [rows, jnp.ones_like(lanes)])
            idxi_v[pl.ds(s * _L, _L)] = ti
            idxk_v[pl.ds(s * _L, _L)] = ki
            return 0

        lax.fori_loop(0, _BW // _L, ik_step, 0, unroll=4)

        jrow = lax.shift_right_logical(lanes, 3)
        jcol = lax.bitwise_and(lanes, 7) + 2

        def j_step(s, _):
            vals = plsc.load_gather(items_v, [jrow + 2 * s, jcol])
            idxj_v[pl.ds(s * _L, _L)] = vals
            return 0

        lax.fori_loop(0, _BW * _NEG // _L, j_step, 0, unroll=8)

        chunks = [
            (idxi_v, 0, vi_hbm, base),
            (idxk_v, 0, vk_hbm, base),
        ] + [
            (idxj_v, c * _CH, vj_hbm, jbase + c * _CH)
            for c in range(_BW * _NEG // _CH)
        ]

        gh, wh = {}, {}
        for t in range(_NCHUNK + 1):
            if t < _NCHUNK:
                b = t % _NBUF
                if t >= _NBUF:
                    wh[t - _NBUF].wait()
                src, ioff, _, _ = chunks[t]
                gh[t] = pltpu.async_copy(
                    table_hbm.at[src.at[pl.ds(ioff, _CH)]], bufs[b], gsem[b])
            u = t - 1
            if 0 <= u < _NCHUNK:
                b = u % _NBUF
                _, _, dst, ooff = chunks[u]
                gh[u].wait()
                wh[u] = pltpu.async_copy(
                    bufs[b], dst.at[pl.ds(ooff, _CH)], wsem[b])
        for u in range(_NCHUNK - _NBUF, _NCHUNK):
            wh[u].wait()

    return body(items, table)


def kernel(items, table):
    vi, vk, vj = _gather_triplets(items.astype(jnp.int32), table)
    return vi, vk, vj.reshape(_B, _NEG, _D)


# speedup vs baseline: 1.0111x; 1.0111x over previous
import functools

import jax
import jax.numpy as jnp
from jax import lax
from jax.experimental import pallas as pl
from jax.experimental.pallas import tpu as pltpu
from jax.experimental.pallas import tpu_sc as plsc

_B = 16384
_D = 64
_NEG = 8
_NC = 2
_NS = 16
_NW = _NC * _NS
_CH = _B // _NW
_NCHUNK = 2 + _NEG
_NBUF = 3


def _gather_triplets(idx_i, idx_k, negs, table):
    mesh = plsc.VectorSubcoreMesh(core_axis_name="c", subcore_axis_name="s")

    @functools.partial(
        pl.kernel,
        mesh=mesh,
        out_type=(
            jax.ShapeDtypeStruct((_B, _D), jnp.float32),
            jax.ShapeDtypeStruct((_B, _D), jnp.float32),
            jax.ShapeDtypeStruct((_B, _NEG, _D), jnp.float32),
        ),
        scratch_types=(
            [
                pltpu.VMEM((_CH,), jnp.int32),
                pltpu.VMEM((_CH,), jnp.int32),
                pltpu.VMEM((_NEG, _CH), jnp.int32),
            ]
            + [pltpu.VMEM((_CH, _D), jnp.float32) for _ in range(_NBUF)]
            + [pltpu.SemaphoreType.DMA for _ in range(2 * _NBUF + 1)]
        ),
        compiler_params=pltpu.CompilerParams(
            use_tc_tiling_on_sc=False, needs_layout_passes=False),
    )
    def body(idx_i_hbm, idx_k_hbm, *rest):
        negs_hbm = rest[:_NEG]
        table_hbm, vi_hbm, vk_hbm, vj_hbm = rest[_NEG:_NEG + 4]
        idxi_v, idxk_v, coln_v = rest[_NEG + 4:_NEG + 7]
        bufs = rest[_NEG + 7:_NEG + 7 + _NBUF]
        gsem = rest[_NEG + 7 + _NBUF:_NEG + 7 + 2 * _NBUF]
        wsem = rest[_NEG + 7 + 2 * _NBUF:_NEG + 7 + 3 * _NBUF]
        isem = rest[_NEG + 7 + 3 * _NBUF]

        wid = lax.axis_index("s") * _NC + lax.axis_index("c")
        base = wid * _CH

        ih = [
            pltpu.async_copy(idx_i_hbm.at[pl.ds(base, _CH)], idxi_v, isem),
            pltpu.async_copy(idx_k_hbm.at[pl.ds(base, _CH)], idxk_v, isem),
        ] + [
            pltpu.async_copy(negs_hbm[c].at[pl.ds(base, _CH)],
                             coln_v.at[c], isem)
            for c in range(_NEG)
        ]
        for h in ih:
            h.wait()

        def out2(dst):
            return lambda buf, sem: pltpu.async_copy(
                buf, dst.at[pl.ds(base, _CH)], sem)

        def out3(c):
            return lambda buf, sem: pltpu.async_copy(
                buf, vj_hbm.at[pl.ds(base, _CH), c], sem)

        chunks = [
            (idxi_v, out2(vi_hbm)),
            (idxk_v, out2(vk_hbm)),
        ] + [
            (coln_v.at[c], out3(c)) for c in range(_NEG)
        ]

        gh, wh = {}, {}
        for t in range(_NCHUNK + 1):
            if t < _NCHUNK:
                b = t % _NBUF
                if t >= _NBUF:
                    wh[t - _NBUF].wait()
                gh[t] = pltpu.async_copy(
                    table_hbm.at[chunks[t][0]], bufs[b], gsem[b])
            u = t - 1
            if 0 <= u < _NCHUNK:
                b = u % _NBUF
                gh[u].wait()
                wh[u] = chunks[u][1](bufs[b], wsem[b])
        for u in range(_NCHUNK - _NBUF, _NCHUNK):
            wh[u].wait()

    return body(idx_i, idx_k, *negs, table)


def kernel(items, table):
    items = items.astype(jnp.int32)
    idx_i = items[:, 0]
    idx_k = items[:, 1]
    negs = [items[:, 2 + c] for c in range(_NEG)]
    return _gather_triplets(idx_i, idx_k, negs, table)
